# Initial kernel scaffold; baseline (speedup 1.0000x reference)
import jax
import jax.numpy as jnp
from jax import lax
from jax.experimental import pallas as pl
from jax.experimental.pallas import tpu as pltpu
from jax.experimental.pallas import tpu_sc as plsc

B = 16384
F = 26
D2 = 16
N = B * F
NC = 2
NS = 16
NW = NC * NS
PER_W = N // NW
CB = 1024
G = CB // 128
NCH = PER_W // CB

C2, C4, C6, C8 = -0.5, 1.0 / 24, -1.0 / 720, 1.0 / 40320
S3, S5, S7, S9 = -1.0 / 6, 1.0 / 120, -1.0 / 5040, 1.0 / 362880


def _body(idx_hbm, phases_hbm, out_hbm, idx_v, rows_v, out_v, sg0, sg1, so0, so1):
    wid = lax.axis_index("s") * NC + lax.axis_index("c")
    row0 = wid * (PER_W // 128)
    obase = wid * (PER_W * 32)

    sgs = (sg0, sg1)
    sos = (so0, so1)

    lane = lax.iota(jnp.int32, (16,))
    offc = lane * 2
    offs = offc + 1

    def start_chunk(g):
        p = g % 2
        pltpu.sync_copy(idx_hbm.at[pl.ds(row0 + g * G, G)], idx_v.at[p])
        return [
            pltpu.async_copy(
                phases_hbm.at[idx_v.at[p, j]],
                rows_v.at[p, pl.ds(j * 128, 128)],
                sgs[p],
            )
            for j in range(G)
        ]

    gathers = {0: start_chunk(0)}
    out_copies = {}
    for g in range(NCH):
        p = g % 2
        if g + 1 < NCH:
            gathers[g + 1] = start_chunk(g + 1)
        for c in gathers.pop(g):
            c.wait()
        if g >= 2:
            out_copies.pop(g - 2).wait()

        @plsc.parallel_loop(0, CB, step=1, unroll=4)
        def _(i, p=p):
            th = rows_v[p, i, :]
            h = th * 0.25
            h2 = h * h
            ch = 1.0 + h2 * (C2 + h2 * (C4 + h2 * (C6 + h2 * C8)))
            sh = h * (1.0 + h2 * (S3 + h2 * (S5 + h2 * (S7 + h2 * S9))))
            ca = 2.0 * ch * ch - 1.0
            sa = 2.0 * sh * ch
            cb = 2.0 * ca * ca - 1.0
            sb = 2.0 * sa * ca
            b32 = i * 32
            plsc.store_scatter(out_v.at[p], [offc + b32], cb)
            plsc.store_scatter(out_v.at[p], [offs + b32], sb)

        out_copies[g] = pltpu.async_copy(
            out_v.at[p],
            out_hbm.at[pl.ds(obase + g * (CB * 32), CB * 32)],
            sos[p],
        )
    for g in sorted(out_copies):
        out_copies[g].wait()


_sc_embed = pl.kernel(
    _body,
    out_type=jax.ShapeDtypeStruct((N * 32,), jnp.float32),
    mesh=plsc.VectorSubcoreMesh(core_axis_name="c", subcore_axis_name="s"),
    scratch_types=[
        pltpu.VMEM((2, G, 128), jnp.int32),
        pltpu.VMEM((2, CB, D2), jnp.float32),
        pltpu.VMEM((2, CB * 32), jnp.float32),
        pltpu.SemaphoreType.DMA,
        pltpu.SemaphoreType.DMA,
        pltpu.SemaphoreType.DMA,
        pltpu.SemaphoreType.DMA,
    ],
)


def kernel(input, phases):
    idx = input.reshape(N).astype(jnp.int32).reshape(N // 128, 128)
    out = _sc_embed(idx, phases)
    return out.reshape(B, F, 2 * D2)


# speedup vs baseline: 3.1035x; 3.1035x over previous
import jax
import jax.numpy as jnp
from jax import lax
from jax.experimental import pallas as pl
from jax.experimental.pallas import tpu as pltpu
from jax.experimental.pallas import tpu_sc as plsc

B = 16384
F = 26
D2 = 16
N = B * F
V = 1_000_000
NC = 2
NS = 16
NW = NC * NS
QF = V // 128
VT = QF * 128
TAIL = V - VT
BPW = B // NW
QW = BPW // 128

C2, C4, C6, C8 = -0.5, 1.0 / 24, -1.0 / 720, 1.0 / 40320
S3, S5, S7, S9 = -1.0 / 6, 1.0 / 120, -1.0 / 5040, 1.0 / 362880


def _detile_body(
    pt_hbm, tail_hbm, tab_hbm, tin, tout0, tout1, tailv, sg0, sg1, so0, so1
):
    tout = (tout0, tout1)
    wid = lax.axis_index("s") * NC + lax.axis_index("c")
    start = wid * 244 + jnp.minimum(wid, 4)
    nblk = 244 + (wid < 4).astype(jnp.int32)

    lane = lax.iota(jnp.int32, 16)
    idxjs = [lane * 16 + j for j in range(16)]
    sgs = (sg0, sg1)
    sos = (so0, so1)

    def rd_refs(t):
        q2 = start + t
        return pt_hbm.at[pl.ds(0, 16), pl.ds(q2 * 128, 128)]

    def wr_refs(t):
        q2 = start + t
        return tab_hbm.at[pl.ds(q2 * 2048, 2048)]

    pltpu.async_copy(rd_refs(0), tin.at[0], sg0)
    pltpu.async_copy(rd_refs(1), tin.at[1], sg1)

    @pl.loop(0, 123)
    def _(g):
        for p in range(2):
            t = 2 * g + p

            @pl.when(t < nblk)
            def _(t=t, p=p):
                pltpu.make_async_copy(rd_refs(t), tin.at[p], sgs[p]).wait()
                for a in range(8):
                    for j in range(16):
                        v = tin[p, j, pl.ds(16 * a, 16)]
                        plsc.store_scatter(tout[p], [idxjs[j] + 256 * a], v)

                @pl.when(t >= 2)
                def _():
                    pltpu.make_async_copy(
                        tout[p], wr_refs(t - 2), sos[p]
                    ).wait()

                pltpu.async_copy(tout[p], wr_refs(t), sos[p])

                @pl.when(t + 2 < nblk)
                def _():
                    pltpu.async_copy(rd_refs(t + 2), tin.at[p], sgs[p])

    for p in range(2):
        pltpu.make_async_copy(
            tout[p], tab_hbm.at[pl.ds(0, 2048)], sos[p]
        ).wait()

    @pl.when(wid == NW - 1)
    def _():
        pltpu.sync_copy(tail_hbm, tailv)
        pltpu.sync_copy(tailv, tab_hbm.at[pl.ds(VT * D2, TAIL * D2)])


_sc_detile = pl.kernel(
    _detile_body,
    out_type=jax.ShapeDtypeStruct((V * D2,), jnp.float32),
    mesh=plsc.VectorSubcoreMesh(core_axis_name="c", subcore_axis_name="s"),
    compiler_params=pltpu.CompilerParams(
        needs_layout_passes=False, use_tc_tiling_on_sc=True
    ),
    scratch_types=[
        pltpu.VMEM((2, 16, 128), jnp.float32),
        pltpu.VMEM((2048,), jnp.float32),
        pltpu.VMEM((2048,), jnp.float32),
        pltpu.VMEM((TAIL * D2,), jnp.float32),
        pltpu.SemaphoreType.DMA,
        pltpu.SemaphoreType.DMA,
        pltpu.SemaphoreType.DMA,
        pltpu.SemaphoreType.DMA,
    ],
)


def _embed_body(
    idx_hbm, tab_hbm, out_hbm, idx_v, rows_v, out_v0, out_v1, sg0, sg1, so0, so1
):
    wid = lax.axis_index("s") * NC + lax.axis_index("c")

    lane = lax.iota(jnp.int32, 16)
    tblc = (lane // 4) * 4096 + (lane % 4) * 256

    sgs = (sg0, sg1)
    sos = (so0, so1)
    outs = (out_v0, out_v1)

    def start_chunk(f):
        p = f % 2
        pltpu.sync_copy(idx_hbm.at[f, pl.ds(QW * wid, QW)], idx_v.at[p])
        return [
            pltpu.async_copy(
                tab_hbm.at[idx_v.at[p, c]],
                rows_v.at[p, pl.ds(c * 128, 128)],
                sgs[p],
            )
            for c in range(QW)
        ]

    gathers = {0: start_chunk(0)}
    out_copies = {}
    for f in range(F):
        p = f % 2
        if f + 1 < F:
            gathers[f + 1] = start_chunk(f + 1)
        for c in gathers.pop(f):
            c.wait()
        if f >= 2:
            for c in out_copies.pop(f - 2):
                c.wait()

        @plsc.parallel_loop(0, BPW, step=1, unroll=4)
        def _(i, p=p):
            th = rows_v[p, i, :]
            h = th * 0.25
            h2 = h * h
            ch = 1.0 + h2 * (C2 + h2 * (C4 + h2 * (C6 + h2 * C8)))
            sh = h * (1.0 + h2 * (S3 + h2 * (S5 + h2 * (S7 + h2 * S9))))
            ca = 2.0 * ch * ch - 1.0
            sa = 2.0 * sh * ch
            cb = 2.0 * ca * ca - 1.0
            sb = 2.0 * sa * ca
            base = 8 * i - 7 * (i & 127)
            idxc = tblc + base
            plsc.store_scatter(outs[p], [idxc], cb)
            plsc.store_scatter(outs[p], [idxc + 128], sb)

        obase = f * (B * 32) + wid * 4096
        out_copies[f] = [
            pltpu.async_copy(
                outs[p].at[pl.ds(kt * 4096, 4096)],
                out_hbm.at[pl.ds(obase + kt * (128 * 1024), 4096)],
                sos[p],
            )
            for kt in range(4)
        ]
    for f in sorted(out_copies):
        for c in out_copies[f]:
            c.wait()


_sc_embed = pl.kernel(
    _embed_body,
    out_type=jax.ShapeDtypeStruct((N * 32,), jnp.float32),
    mesh=plsc.VectorSubcoreMesh(core_axis_name="c", subcore_axis_name="s"),
    compiler_params=pltpu.CompilerParams(
        needs_layout_passes=False, use_tc_tiling_on_sc=False
    ),
    scratch_types=[
        pltpu.VMEM((2, QW, 128), jnp.int32),
        pltpu.VMEM((2, BPW, D2), jnp.float32),
        pltpu.VMEM((32 * 512,), jnp.float32),
        pltpu.VMEM((32 * 512,), jnp.float32),
        pltpu.SemaphoreType.DMA,
        pltpu.SemaphoreType.DMA,
        pltpu.SemaphoreType.DMA,
        pltpu.SemaphoreType.DMA,
    ],
)


def kernel(input, phases):
    phases_t = phases.T
    tail = phases[VT:, :].reshape(TAIL * D2)
    table = _sc_detile(phases_t, tail)
    idx3 = input.T.reshape(F, 128, 128).astype(jnp.int32)
    flat = _sc_embed(idx3, table.reshape(V, D2))
    out = flat.reshape(F, 4, 128, 8, 128).transpose(2, 4, 0, 1, 3)
    return out.reshape(B, F, 32)
